# baseline (speedup 1.0000x reference)
import functools

import jax
import jax.numpy as jnp
from jax import lax
from jax.experimental import pallas as pl
from jax.experimental.pallas import tpu as pltpu
from jax.experimental.pallas import tpu_sc as plsc

B, M, D = 128, 1024, 768
H = D // 4
K = 256
BM = 4096
NV = M // 16
L = 16

_NEG_INF = float("-inf")


def _mlp_body(x_ref, w1_ref, b1_ref, w2_ref, b2_ref, out_ref):
    x = x_ref[...]
    hid = lax.dot_general(x, w1_ref[...], (((1,), (0,)), ((), ())),
                          preferred_element_type=jnp.float32)
    hid = jnp.maximum(hid + b1_ref[...], 0.0)
    logits = lax.dot_general(hid, w2_ref[...], (((1,), (0,)), ((), ())),
                             preferred_element_type=jnp.float32)
    out_ref[...] = logits + b2_ref[...]


def _mlp_logits(x2d, W1, b1, W2, b2):
    n = x2d.shape[0]
    w2p = jnp.pad(W2, ((0, 0), (0, 127)))
    out = pl.pallas_call(
        _mlp_body,
        grid=(n // BM,),
        in_specs=[
            pl.BlockSpec((BM, D), lambda i: (i, 0)),
            pl.BlockSpec((D, H), lambda i: (0, 0)),
            pl.BlockSpec((1, H), lambda i: (0, 0)),
            pl.BlockSpec((H, 128), lambda i: (0, 0)),
            pl.BlockSpec((1, 128), lambda i: (0, 0)),
        ],
        out_specs=pl.BlockSpec((BM, 128), lambda i: (i, 0)),
        out_shape=jax.ShapeDtypeStruct((n, 128), jnp.float32),
    )(x2d, W1, b1.reshape(1, H), w2p,
      jnp.pad(b2.reshape(1, 1), ((0, 0), (0, 127))))
    return out[:, 0]


def _splat(x):
    return jnp.full((L,), x, jnp.float32)


_GDIMS = lax.GatherDimensionNumbers(
    offset_dims=(), collapsed_slice_dims=(0,), start_index_map=(0,))


def _permute(v, p):
    return lax.gather(v, p[:, None], _GDIMS, (1,),
                      mode=lax.GatherScatterMode.PROMISE_IN_BOUNDS)


def _bfly_max(v, perms):
    for p in perms:
        v = jnp.maximum(v, _permute(v, p))
    return v


def _bfly_min(v, perms):
    for p in perms:
        v = jnp.minimum(v, _permute(v, p))
    return v


def _scal(x):
    return x if getattr(x, "ndim", 0) == 0 else x[0]


@functools.lru_cache(maxsize=4)
def _sc_topk_build(nrows=B):
    NC, NS = 2, 16
    NW = NC * NS
    rows_per_w = nrows // NW
    mesh = plsc.VectorSubcoreMesh(core_axis_name="c", subcore_axis_name="s")

    @functools.partial(
        pl.kernel,
        mesh=mesh,
        out_type=[
            jax.ShapeDtypeStruct((nrows * M,), jnp.float32),
            jax.ShapeDtypeStruct((nrows * K,), jnp.int32),
        ],
        scratch_types=(
            [pltpu.VMEM((M,), jnp.float32) for _ in range(rows_per_w)]
            + [pltpu.VMEM((M,), jnp.float32) for _ in range(rows_per_w)]
            + [pltpu.VMEM((NV,), jnp.float32) for _ in range(rows_per_w)]
            + [pltpu.VMEM((K,), jnp.int32) for _ in range(rows_per_w)]
        ),
    )
    def sc_topk(logits_hbm, mask_hbm, idx_hbm, *scr):
        rw = rows_per_w
        vrows = scr[0:rw]
        vmasks = scr[rw:2 * rw]
        pvs = scr[2 * rw:3 * rw]
        vidxs = scr[3 * rw:4 * rw]
        wid = lax.axis_index("s") * NC + lax.axis_index("c")
        lanes = lax.iota(jnp.int32, L)
        zeros16 = jnp.zeros((L,), jnp.float32)
        perms = [lanes ^ 1, lanes ^ 2, lanes ^ 4, lanes ^ 8]
        RW = rows_per_w

        for rr in range(RW):
            row = wid * RW + rr
            pltpu.sync_copy(logits_hbm.at[pl.ds(row * M, M)], vrows[rr])
            pltpu.sync_copy(logits_hbm.at[pl.ds(row * M, M)], vmasks[rr])

        for q in range(NV // L):
            chunks = [zeros16] * RW
            for l in range(L):
                jv = q * L + l
                for rr in range(RW):
                    ms = _bfly_max(vrows[rr][pl.ds(jv * L, L)], perms)
                    chunks[rr] = jnp.where(lanes == l, ms, chunks[rr])
            for rr in range(RW):
                pvs[rr][pl.ds(q * L, L)] = chunks[rr]

        big = jnp.full((L,), NV, jnp.int32)
        big16 = jnp.full((L,), L, jnp.int32)
        neginf = _splat(_NEG_INF)
        UNROLL = 4

        def one_extract(t, rr):
            vrow, pv, vidx = vrows[rr], pvs[rr], vidxs[rr]
            c0 = pv[pl.ds(0, L)]
            c1 = pv[pl.ds(L, L)]
            c2 = pv[pl.ds(2 * L, L)]
            c3 = pv[pl.ds(3 * L, L)]
            gs = _bfly_max(jnp.maximum(jnp.maximum(c0, c1),
                                       jnp.maximum(c2, c3)), perms)
            cand = jnp.minimum(
                jnp.minimum(jnp.where(c0 == gs, lanes, big),
                            jnp.where(c1 == gs, lanes + L, big)),
                jnp.minimum(jnp.where(c2 == gs, lanes + 2 * L, big),
                            jnp.where(c3 == gs, lanes + 3 * L, big)))
            js = _bfly_min(cand, perms)
            jstar = _scal(js)
            v = vrow[pl.ds(jstar * L, L)]
            lv = _bfly_min(jnp.where(v == gs, lanes, big16), perms)
            idxv = js * L + lv
            tc = t // L
            tl = t % L
            ich = vidx[pl.ds(tc * L, L)]
            vidx[pl.ds(tc * L, L)] = jnp.where(lanes == tl, idxv, ich)
            vnew = jnp.where(lanes == lv, neginf, v)
            vrow[pl.ds(jstar * L, L)] = vnew
            nms = _bfly_max(vnew, perms)
            lq = js & (L - 1)
            pch = pv[pl.ds((jstar // L) * L, L)]
            pv[pl.ds((jstar // L) * L, L)] = jnp.where(lanes == lq, nms, pch)

        def extract(i, _):
            for u in range(UNROLL):
                t = i * UNROLL + u
                for rr in range(RW):
                    one_extract(t, rr)
            return 0

        lax.fori_loop(0, K // UNROLL, extract, 0)

        one = jnp.float32(1.0)
        for j in range(NV):
            for rr in range(RW):
                a = vmasks[rr][pl.ds(j * L, L)]
                sel = vrows[rr][pl.ds(j * L, L)] == neginf
                vmasks[rr][pl.ds(j * L, L)] = jnp.where(
                    sel, (one - a) + a, jnp.zeros((L,), jnp.float32))

        for rr in range(RW):
            row = wid * RW + rr
            pltpu.sync_copy(vmasks[rr], mask_hbm.at[pl.ds(row * M, M)])
            pltpu.sync_copy(vidxs[rr], idx_hbm.at[pl.ds(row * K, K)])

    return sc_topk


NCHUNK = 2


def kernel(patch_embeddings, W1, b1, W2, b2):
    Bc, Mc, Dc = patch_embeddings.shape
    x2d = patch_embeddings.reshape(Bc * Mc, Dc)
    bch = Bc // NCHUNK
    sc = _sc_topk_build(bch)
    logit_chunks = [
        _mlp_logits(x2d[c * bch * Mc:(c + 1) * bch * Mc], W1, b1, W2, b2)
        for c in range(NCHUNK)
    ]
    outs = [sc(lf) for lf in logit_chunks]
    logits = jnp.concatenate(logit_chunks).reshape(Bc, Mc)
    mask = jnp.concatenate([o[0] for o in outs]).reshape(Bc, Mc)
    topk_indices = jnp.concatenate([o[1] for o in outs]).reshape(Bc, K)
    return (mask, logits, topk_indices)


# speedup vs baseline: 2.0476x; 2.0476x over previous
import functools

import jax
import jax.numpy as jnp
from jax import lax
from jax.experimental import pallas as pl
from jax.experimental.pallas import tpu as pltpu
from jax.experimental.pallas import tpu_sc as plsc

B, M, D = 128, 1024, 768
H = D // 4
K = 256
BM = 4096
NV = M // 16
L = 16

_NEG_INF = float("-inf")


def _mlp_body(x_ref, w1_ref, b1_ref, w2_ref, b2_ref, out_ref):
    x = x_ref[...]
    hid = lax.dot_general(x, w1_ref[...], (((1,), (0,)), ((), ())),
                          preferred_element_type=jnp.float32)
    hid = jnp.maximum(hid + b1_ref[...], 0.0)
    logits = lax.dot_general(hid, w2_ref[...], (((1,), (0,)), ((), ())),
                             preferred_element_type=jnp.float32)
    out_ref[...] = logits + b2_ref[...]


def _mlp_logits(x2d, W1, b1, W2, b2, base=0, nrows=None):
    n = nrows if nrows is not None else x2d.shape[0]
    w2p = jnp.pad(W2, ((0, 0), (0, 127)))
    out = pl.pallas_call(
        _mlp_body,
        grid=(n // BM,),
        in_specs=[
            pl.BlockSpec((BM, D), lambda i: (i + base, 0)),
            pl.BlockSpec((D, H), lambda i: (0, 0)),
            pl.BlockSpec((1, H), lambda i: (0, 0)),
            pl.BlockSpec((H, 128), lambda i: (0, 0)),
            pl.BlockSpec((1, 128), lambda i: (0, 0)),
        ],
        out_specs=pl.BlockSpec((BM, 128), lambda i: (i, 0)),
        out_shape=jax.ShapeDtypeStruct((n, 128), jnp.float32),
    )(x2d, W1, b1.reshape(1, H), w2p,
      jnp.pad(b2.reshape(1, 1), ((0, 0), (0, 127))))
    return out[:, 0]


def _splat(x):
    return jnp.full((L,), x, jnp.float32)


_GDIMS = lax.GatherDimensionNumbers(
    offset_dims=(), collapsed_slice_dims=(0,), start_index_map=(0,))


def _permute(v, p):
    return lax.gather(v, p[:, None], _GDIMS, (1,),
                      mode=lax.GatherScatterMode.PROMISE_IN_BOUNDS)


def _bfly_max(v, perms):
    for p in perms:
        v = jnp.maximum(v, _permute(v, p))
    return v


def _bfly_min(v, perms):
    for p in perms:
        v = jnp.minimum(v, _permute(v, p))
    return v


def _scal(x):
    return x if getattr(x, "ndim", 0) == 0 else x[0]


@functools.lru_cache(maxsize=4)
def _sc_topk_build(nrows=B):
    NC, NS = 2, 16
    NW = NC * NS
    rows_per_w = nrows // NW
    mesh = plsc.VectorSubcoreMesh(core_axis_name="c", subcore_axis_name="s")

    @functools.partial(
        pl.kernel,
        mesh=mesh,
        out_type=[
            jax.ShapeDtypeStruct((nrows * M,), jnp.float32),
            jax.ShapeDtypeStruct((nrows * K,), jnp.int32),
        ],
        scratch_types=(
            [pltpu.VMEM((M,), jnp.float32) for _ in range(rows_per_w)]
            + [pltpu.VMEM((M,), jnp.float32) for _ in range(rows_per_w)]
            + [pltpu.VMEM((NV,), jnp.float32) for _ in range(rows_per_w)]
            + [pltpu.VMEM((K,), jnp.int32) for _ in range(rows_per_w)]
        ),
    )
    def sc_topk(logits_hbm, mask_hbm, idx_hbm, *scr):
        rw = rows_per_w
        vrows = scr[0:rw]
        vmasks = scr[rw:2 * rw]
        pvs = scr[2 * rw:3 * rw]
        vidxs = scr[3 * rw:4 * rw]
        wid = lax.axis_index("s") * NC + lax.axis_index("c")
        lanes = lax.iota(jnp.int32, L)
        zeros16 = jnp.zeros((L,), jnp.float32)
        perms = [lanes ^ 1, lanes ^ 2, lanes ^ 4, lanes ^ 8]
        RW = rows_per_w

        for rr in range(RW):
            row = wid * RW + rr
            pltpu.sync_copy(logits_hbm.at[pl.ds(row * M, M)], vrows[rr])
            pltpu.sync_copy(logits_hbm.at[pl.ds(row * M, M)], vmasks[rr])

        for q in range(NV // L):
            chunks = [zeros16] * RW
            for l in range(L):
                jv = q * L + l
                for rr in range(RW):
                    ms = _bfly_max(vrows[rr][pl.ds(jv * L, L)], perms)
                    chunks[rr] = jnp.where(lanes == l, ms, chunks[rr])
            for rr in range(RW):
                pvs[rr][pl.ds(q * L, L)] = chunks[rr]

        big = jnp.full((L,), NV, jnp.int32)
        big16 = jnp.full((L,), L, jnp.int32)
        neginf = _splat(_NEG_INF)
        UNROLL = 4

        def one_extract(t, rr):
            vrow, pv, vidx = vrows[rr], pvs[rr], vidxs[rr]
            c0 = pv[pl.ds(0, L)]
            c1 = pv[pl.ds(L, L)]
            c2 = pv[pl.ds(2 * L, L)]
            c3 = pv[pl.ds(3 * L, L)]
            gs = _bfly_max(jnp.maximum(jnp.maximum(c0, c1),
                                       jnp.maximum(c2, c3)), perms)
            cand = jnp.minimum(
                jnp.minimum(jnp.where(c0 == gs, lanes, big),
                            jnp.where(c1 == gs, lanes + L, big)),
                jnp.minimum(jnp.where(c2 == gs, lanes + 2 * L, big),
                            jnp.where(c3 == gs, lanes + 3 * L, big)))
            js = _bfly_min(cand, perms)
            jstar = _scal(js)
            v = vrow[pl.ds(jstar * L, L)]
            lv = _bfly_min(jnp.where(v == gs, lanes, big16), perms)
            idxv = js * L + lv
            tc = t // L
            tl = t % L
            ich = vidx[pl.ds(tc * L, L)]
            vidx[pl.ds(tc * L, L)] = jnp.where(lanes == tl, idxv, ich)
            vnew = jnp.where(lanes == lv, neginf, v)
            vrow[pl.ds(jstar * L, L)] = vnew
            nms = _bfly_max(vnew, perms)
            lq = js & (L - 1)
            pch = pv[pl.ds((jstar // L) * L, L)]
            pv[pl.ds((jstar // L) * L, L)] = jnp.where(lanes == lq, nms, pch)

        def extract(i, _):
            for u in range(UNROLL):
                t = i * UNROLL + u
                for rr in range(RW):
                    one_extract(t, rr)
            return 0

        lax.fori_loop(0, K // UNROLL, extract, 0)

        one = jnp.float32(1.0)
        for j in range(NV):
            for rr in range(RW):
                a = vmasks[rr][pl.ds(j * L, L)]
                sel = vrows[rr][pl.ds(j * L, L)] == neginf
                vmasks[rr][pl.ds(j * L, L)] = jnp.where(
                    sel, (one - a) + a, jnp.zeros((L,), jnp.float32))

        for rr in range(RW):
            row = wid * RW + rr
            pltpu.sync_copy(vmasks[rr], mask_hbm.at[pl.ds(row * M, M)])
            pltpu.sync_copy(vidxs[rr], idx_hbm.at[pl.ds(row * K, K)])

    return sc_topk


NCHUNK = 2


def kernel(patch_embeddings, W1, b1, W2, b2):
    Bc, Mc, Dc = patch_embeddings.shape
    x2d = patch_embeddings.reshape(Bc * Mc, Dc)
    bch = Bc // NCHUNK
    sc = _sc_topk_build(bch)
    cb = bch * Mc // BM
    logit_chunks = [
        _mlp_logits(x2d, W1, b1, W2, b2, base=c * cb, nrows=bch * Mc)
        for c in range(NCHUNK)
    ]
    outs = [sc(lf) for lf in logit_chunks]
    logits = jnp.concatenate(logit_chunks).reshape(Bc, Mc)
    mask = jnp.concatenate([o[0] for o in outs]).reshape(Bc, Mc)
    topk_indices = jnp.concatenate([o[1] for o in outs]).reshape(Bc, K)
    return (mask, logits, topk_indices)


# speedup vs baseline: 2.0774x; 1.0145x over previous
import functools

import jax
import jax.numpy as jnp
from jax import lax
from jax.experimental import pallas as pl
from jax.experimental.pallas import tpu as pltpu
from jax.experimental.pallas import tpu_sc as plsc

B, M, D = 128, 1024, 768
H = D // 4
K = 256
BM = 4096
NV = M // 16
L = 16

_NEG_INF = float("-inf")


def _mlp_body(x_ref, w1_ref, b1_ref, w2_ref, b2_ref, out_ref):
    x = x_ref[...]
    hid = lax.dot_general(x, w1_ref[...], (((1,), (0,)), ((), ())),
                          preferred_element_type=jnp.float32)
    hid = jnp.maximum(hid + b1_ref[...], 0.0)
    logits = lax.dot_general(hid, w2_ref[...], (((1,), (0,)), ((), ())),
                             preferred_element_type=jnp.float32)
    out_ref[...] = logits + b2_ref[...]


def _mlp_logits(x2d, W1, b1, W2, b2, base=0, nrows=None):
    n = nrows if nrows is not None else x2d.shape[0]
    w2p = jnp.pad(W2, ((0, 0), (0, 127)))
    out = pl.pallas_call(
        _mlp_body,
        grid=(n // BM,),
        in_specs=[
            pl.BlockSpec((BM, D), lambda i: (i + base, 0)),
            pl.BlockSpec((D, H), lambda i: (0, 0)),
            pl.BlockSpec((1, H), lambda i: (0, 0)),
            pl.BlockSpec((H, 128), lambda i: (0, 0)),
            pl.BlockSpec((1, 128), lambda i: (0, 0)),
        ],
        out_specs=pl.BlockSpec((BM, 128), lambda i: (i, 0)),
        out_shape=jax.ShapeDtypeStruct((n, 128), jnp.float32),
    )(x2d, W1, b1.reshape(1, H), w2p,
      jnp.pad(b2.reshape(1, 1), ((0, 0), (0, 127))))
    return out[:, 0]


def _splat(x):
    return jnp.full((L,), x, jnp.float32)


_GDIMS = lax.GatherDimensionNumbers(
    offset_dims=(), collapsed_slice_dims=(0,), start_index_map=(0,))


def _permute(v, p):
    return lax.gather(v, p[:, None], _GDIMS, (1,),
                      mode=lax.GatherScatterMode.PROMISE_IN_BOUNDS)


def _bfly_max(v, perms):
    for p in perms:
        v = jnp.maximum(v, _permute(v, p))
    return v


def _bfly_min(v, perms):
    for p in perms:
        v = jnp.minimum(v, _permute(v, p))
    return v


def _scal(x):
    return x if getattr(x, "ndim", 0) == 0 else x[0]


@functools.lru_cache(maxsize=4)
def _sc_topk_build(nrows=B):
    NC, NS = 2, 16
    NW = NC * NS
    rows_per_w = nrows // NW
    mesh = plsc.VectorSubcoreMesh(core_axis_name="c", subcore_axis_name="s")

    @functools.partial(
        pl.kernel,
        mesh=mesh,
        out_type=[
            jax.ShapeDtypeStruct((nrows * M,), jnp.float32),
            jax.ShapeDtypeStruct((nrows * K,), jnp.int32),
        ],
        scratch_types=(
            [pltpu.VMEM((M,), jnp.float32) for _ in range(rows_per_w)]
            + [pltpu.VMEM((M,), jnp.float32) for _ in range(rows_per_w)]
            + [pltpu.VMEM((NV,), jnp.float32) for _ in range(rows_per_w)]
            + [pltpu.VMEM((K,), jnp.int32) for _ in range(rows_per_w)]
        ),
    )
    def sc_topk(logits_hbm, mask_hbm, idx_hbm, *scr):
        rw = rows_per_w
        vrows = scr[0:rw]
        vmasks = scr[rw:2 * rw]
        pvs = scr[2 * rw:3 * rw]
        vidxs = scr[3 * rw:4 * rw]
        wid = lax.axis_index("s") * NC + lax.axis_index("c")
        lanes = lax.iota(jnp.int32, L)
        zeros16 = jnp.zeros((L,), jnp.float32)
        perms = [lanes ^ 1, lanes ^ 2, lanes ^ 4, lanes ^ 8]
        RW = rows_per_w

        for rr in range(RW):
            row = wid * RW + rr
            pltpu.sync_copy(logits_hbm.at[pl.ds(row * M, M)], vrows[rr])
            pltpu.sync_copy(logits_hbm.at[pl.ds(row * M, M)], vmasks[rr])

        for q in range(NV // L):
            chunks = [zeros16] * RW
            for l in range(L):
                jv = q * L + l
                for rr in range(RW):
                    ms = _bfly_max(vrows[rr][pl.ds(jv * L, L)], perms)
                    chunks[rr] = jnp.where(lanes == l, ms, chunks[rr])
            for rr in range(RW):
                pvs[rr][pl.ds(q * L, L)] = chunks[rr]

        big = jnp.full((L,), NV, jnp.int32)
        big16 = jnp.full((L,), L, jnp.int32)
        neginf = _splat(_NEG_INF)
        UNROLL = 4

        def one_extract(t, rr):
            vrow, pv, vidx = vrows[rr], pvs[rr], vidxs[rr]
            c0 = pv[pl.ds(0, L)]
            c1 = pv[pl.ds(L, L)]
            c2 = pv[pl.ds(2 * L, L)]
            c3 = pv[pl.ds(3 * L, L)]
            gs = _bfly_max(jnp.maximum(jnp.maximum(c0, c1),
                                       jnp.maximum(c2, c3)), perms)
            cand = jnp.minimum(
                jnp.minimum(jnp.where(c0 == gs, lanes, big),
                            jnp.where(c1 == gs, lanes + L, big)),
                jnp.minimum(jnp.where(c2 == gs, lanes + 2 * L, big),
                            jnp.where(c3 == gs, lanes + 3 * L, big)))
            js = _bfly_min(cand, perms)
            jstar = _scal(js)
            v = vrow[pl.ds(jstar * L, L)]
            lv = _bfly_min(jnp.where(v == gs, lanes, big16), perms)
            idxv = js * L + lv
            tc = t // L
            tl = t % L
            ich = vidx[pl.ds(tc * L, L)]
            vidx[pl.ds(tc * L, L)] = jnp.where(lanes == tl, idxv, ich)
            vnew = jnp.where(lanes == lv, neginf, v)
            vrow[pl.ds(jstar * L, L)] = vnew
            nms = _bfly_max(vnew, perms)
            lq = js & (L - 1)
            pch = pv[pl.ds((jstar // L) * L, L)]
            pv[pl.ds((jstar // L) * L, L)] = jnp.where(lanes == lq, nms, pch)

        def extract(i, _):
            for u in range(UNROLL):
                t = i * UNROLL + u
                for rr in range(RW):
                    one_extract(t, rr)
            return 0

        lax.fori_loop(0, K // UNROLL, extract, 0)

        one = jnp.float32(1.0)
        for j in range(NV):
            for rr in range(RW):
                a = vmasks[rr][pl.ds(j * L, L)]
                sel = vrows[rr][pl.ds(j * L, L)] == neginf
                vmasks[rr][pl.ds(j * L, L)] = jnp.where(
                    sel, (one - a) + a, jnp.zeros((L,), jnp.float32))

        for rr in range(RW):
            row = wid * RW + rr
            pltpu.sync_copy(vmasks[rr], mask_hbm.at[pl.ds(row * M, M)])
            pltpu.sync_copy(vidxs[rr], idx_hbm.at[pl.ds(row * K, K)])

    return sc_topk


NCHUNK = 4


def kernel(patch_embeddings, W1, b1, W2, b2):
    Bc, Mc, Dc = patch_embeddings.shape
    x2d = patch_embeddings.reshape(Bc * Mc, Dc)
    bch = Bc // NCHUNK
    sc = _sc_topk_build(bch)
    cb = bch * Mc // BM
    logit_chunks = [
        _mlp_logits(x2d, W1, b1, W2, b2, base=c * cb, nrows=bch * Mc)
        for c in range(NCHUNK)
    ]
    outs = [sc(lf) for lf in logit_chunks]
    logits = jnp.concatenate(logit_chunks).reshape(Bc, Mc)
    mask = jnp.concatenate([o[0] for o in outs]).reshape(Bc, Mc)
    topk_indices = jnp.concatenate([o[1] for o in outs]).reshape(Bc, K)
    return (mask, logits, topk_indices)
